# initial kernel scaffold (unmeasured)
import jax
import jax.numpy as jnp
from jax import lax
from jax.experimental import pallas as pl
from jax.experimental.pallas import tpu as pltpu

N_DEV = 8
B, H, D, BS = 8, 8, 64, 16
PAGES = 64
NKEY = PAGES * BS
PACK = 128
NEG = -1e30


def kernel(Q, K, V, bt, lens):
    lens2 = lens.reshape(B, 1)

    def body(q_ref, k_ref, v_ref, bt_ref, lens_ref, out_ref,
             part_ref, buf_ref, send_sems, recv_sems):
        me = lax.axis_index("i")

        bsem = pltpu.get_barrier_semaphore()
        for p in range(N_DEV):
            @pl.when(p != me)
            def _():
                pl.semaphore_signal(
                    bsem, inc=1,
                    device_id=(p,), device_id_type=pl.DeviceIdType.MESH,
                )
        pl.semaphore_wait(bsem, N_DEV - 1)

        q = q_ref[...].reshape(B, H, D)
        k2 = k_ref[...].reshape(NKEY, H, D)
        v2 = v_ref[...].reshape(NKEY, H, D)

        local_id = bt_ref[...] - me * PAGES
        slot_iota = lax.broadcasted_iota(jnp.int32, (B, 64), 1)
        valid_slot = slot_iota < lens_ref[...]
        key_page = lax.broadcasted_iota(jnp.int32, (B, 64, NKEY), 2) // BS
        hit = (local_id[:, :, None] == key_page) & valid_slot[:, :, None]
        countk = jnp.sum(hit.astype(jnp.float32), axis=1)
        mask = countk > 0.0

        scale = D ** -0.5
        for h in range(H):
            qh = q[:, h, :].astype(jnp.bfloat16)
            kh = k2[:, h, :].astype(jnp.bfloat16)
            sh = lax.dot_general(
                qh, kh, (((1,), (1,)), ((), ())),
                preferred_element_type=jnp.float32,
            ) * scale
            mh = jnp.max(jnp.where(mask, sh, NEG), axis=1, keepdims=True)
            eh = jnp.where(mask, countk * jnp.exp(sh - mh), 0.0)
            lh = jnp.sum(eh, axis=1, keepdims=True)
            vh = v2[:, h, :].astype(jnp.bfloat16)
            oh = lax.dot_general(
                eh.astype(jnp.bfloat16), vh, (((1,), (0,)), ((), ())),
                preferred_element_type=jnp.float32,
            )
            part_ref[:, h, 0:D] = oh
            part_ref[:, h, D:D + 1] = mh
            part_ref[:, h, D + 1:D + 2] = lh

        sends = []
        for p in range(N_DEV):
            rdma = pltpu.make_async_remote_copy(
                src_ref=part_ref,
                dst_ref=buf_ref.at[me],
                send_sem=send_sems.at[p],
                recv_sem=recv_sems.at[me],
                device_id=(p,),
                device_id_type=pl.DeviceIdType.MESH,
            )
            sends.append(rdma)

            @pl.when(p != me)
            def _(rdma=rdma):
                rdma.start()

        for p in range(N_DEV):
            recv = pltpu.make_async_remote_copy(
                src_ref=part_ref,
                dst_ref=buf_ref.at[p],
                send_sem=send_sems.at[p],
                recv_sem=recv_sems.at[p],
                device_id=(p,),
                device_id_type=pl.DeviceIdType.MESH,
            )

            @pl.when(p != me)
            def _(recv=recv):
                recv.wait_recv()

        my = part_ref[...]
        my_o = my[:, :, 0:D]
        my_m = my[:, :, D:D + 1]
        my_l = my[:, :, D + 1:D + 2]

        os_, ms_, ls_ = [], [], []
        for p in range(N_DEV):
            bp = buf_ref[p]
            is_me = p == me
            os_.append(jnp.where(is_me, my_o, bp[:, :, 0:D]))
            ms_.append(jnp.where(is_me, my_m, bp[:, :, D:D + 1]))
            ls_.append(jnp.where(is_me, my_l, bp[:, :, D + 1:D + 2]))

        gmax = ms_[0]
        for p in range(1, N_DEV):
            gmax = jnp.maximum(gmax, ms_[p])
        num = jnp.zeros((B, H, D), jnp.float32)
        den = jnp.zeros((B, H, 1), jnp.float32)
        for p in range(N_DEV):
            w = jnp.exp(ms_[p] - gmax)
            num = num + w * os_[p]
            den = den + w * ls_[p]
        out_ref[...] = (num / den).reshape(B, 1, H, D)

        for p in range(N_DEV):
            @pl.when(p != me)
            def _(rdma=sends[p]):
                rdma.wait_send()

    return pl.pallas_call(
        body,
        out_shape=jax.ShapeDtypeStruct((B, 1, H, D), jnp.float32),
        in_specs=[pl.BlockSpec(memory_space=pltpu.VMEM)] * 5,
        out_specs=pl.BlockSpec(memory_space=pltpu.VMEM),
        scratch_shapes=[
            pltpu.VMEM((B, H, PACK), jnp.float32),
            pltpu.VMEM((N_DEV, B, H, PACK), jnp.float32),
            pltpu.SemaphoreType.DMA((N_DEV,)),
            pltpu.SemaphoreType.DMA((N_DEV,)),
        ],
        compiler_params=pltpu.CompilerParams(collective_id=0),
    )(Q, K, V, bt, lens2)


# baseline (device time: 21897 ns/iter reference)
import jax
import jax.numpy as jnp
from jax import lax
from jax.experimental import pallas as pl
from jax.experimental.pallas import tpu as pltpu

N_DEV = 8
B, H, D, BS = 8, 8, 64, 16
PAGES = 64
NKEY = PAGES * BS
PACK = 128
NEG = -1e30


def kernel(Q, K, V, bt, lens):
    lens2 = lens.reshape(B, 1)

    def body(q_ref, k_ref, v_ref, bt_ref, lens_ref, out_ref,
             part_ref, buf_ref, send_sems, recv_sems):
        me = lax.axis_index("i")

        bsem = pltpu.get_barrier_semaphore()
        for p in range(N_DEV):
            @pl.when(p != me)
            def _():
                pl.semaphore_signal(
                    bsem, inc=1,
                    device_id=(p,), device_id_type=pl.DeviceIdType.MESH,
                )
        pl.semaphore_wait(bsem, N_DEV - 1)

        q = q_ref[...].reshape(B, H, D)
        k2 = k_ref[...].reshape(NKEY, H, D)
        v2 = v_ref[...].reshape(NKEY, H, D)

        local_id = bt_ref[...] - me * PAGES
        slot_iota = lax.broadcasted_iota(jnp.int32, (B, 64), 1)
        valid_slot = slot_iota < lens_ref[...]
        local2 = jnp.where(valid_slot, local_id, -1)
        cols = []
        for p in range(PAGES):
            cols.append(jnp.sum((local2 == p).astype(jnp.float32),
                                axis=1, keepdims=True))
        countp = jnp.concatenate(cols, axis=1)
        expand = (
            lax.broadcasted_iota(jnp.int32, (PAGES, NKEY), 0)
            == lax.broadcasted_iota(jnp.int32, (PAGES, NKEY), 1) // BS
        ).astype(jnp.float32)
        countk = lax.dot_general(
            countp, expand, (((1,), (0,)), ((), ())),
            preferred_element_type=jnp.float32,
        )
        mask = countk > 0.0

        scale = D ** -0.5
        for h in range(H):
            qh = q[:, h, :].astype(jnp.bfloat16)
            kh = k2[:, h, :].astype(jnp.bfloat16)
            sh = lax.dot_general(
                qh, kh, (((1,), (1,)), ((), ())),
                preferred_element_type=jnp.float32,
            ) * scale
            mh = jnp.max(jnp.where(mask, sh, NEG), axis=1, keepdims=True)
            eh = jnp.where(mask, countk * jnp.exp(sh - mh), 0.0)
            lh = jnp.sum(eh, axis=1, keepdims=True)
            vh = v2[:, h, :].astype(jnp.bfloat16)
            oh = lax.dot_general(
                eh.astype(jnp.bfloat16), vh, (((1,), (0,)), ((), ())),
                preferred_element_type=jnp.float32,
            )
            part_ref[:, h, 0:D] = oh
            part_ref[:, h, D:D + 1] = mh
            part_ref[:, h, D + 1:D + 2] = lh

        sends = []
        for p in range(N_DEV):
            rdma = pltpu.make_async_remote_copy(
                src_ref=part_ref,
                dst_ref=buf_ref.at[me],
                send_sem=send_sems.at[p],
                recv_sem=recv_sems.at[me],
                device_id=(p,),
                device_id_type=pl.DeviceIdType.MESH,
            )
            sends.append(rdma)

            @pl.when(p != me)
            def _(rdma=rdma):
                rdma.start()

        for p in range(N_DEV):
            recv = pltpu.make_async_remote_copy(
                src_ref=part_ref,
                dst_ref=buf_ref.at[p],
                send_sem=send_sems.at[p],
                recv_sem=recv_sems.at[p],
                device_id=(p,),
                device_id_type=pl.DeviceIdType.MESH,
            )

            @pl.when(p != me)
            def _(recv=recv):
                recv.wait_recv()

        my = part_ref[...]
        my_o = my[:, :, 0:D]
        my_m = my[:, :, D:D + 1]
        my_l = my[:, :, D + 1:D + 2]

        os_, ms_, ls_ = [], [], []
        for p in range(N_DEV):
            bp = buf_ref[p]
            is_me = p == me
            os_.append(jnp.where(is_me, my_o, bp[:, :, 0:D]))
            ms_.append(jnp.where(is_me, my_m, bp[:, :, D:D + 1]))
            ls_.append(jnp.where(is_me, my_l, bp[:, :, D + 1:D + 2]))

        gmax = ms_[0]
        for p in range(1, N_DEV):
            gmax = jnp.maximum(gmax, ms_[p])
        num = jnp.zeros((B, H, D), jnp.float32)
        den = jnp.zeros((B, H, 1), jnp.float32)
        for p in range(N_DEV):
            w = jnp.exp(ms_[p] - gmax)
            num = num + w * os_[p]
            den = den + w * ls_[p]
        out_ref[...] = (num / den).reshape(B, 1, H, D)

        for p in range(N_DEV):
            @pl.when(p != me)
            def _(rdma=sends[p]):
                rdma.wait_send()

    return pl.pallas_call(
        body,
        out_shape=jax.ShapeDtypeStruct((B, 1, H, D), jnp.float32),
        in_specs=[pl.BlockSpec(memory_space=pltpu.VMEM)] * 5,
        out_specs=pl.BlockSpec(memory_space=pltpu.VMEM),
        scratch_shapes=[
            pltpu.VMEM((B, H, PACK), jnp.float32),
            pltpu.VMEM((N_DEV, B, H, PACK), jnp.float32),
            pltpu.SemaphoreType.DMA((N_DEV,)),
            pltpu.SemaphoreType.DMA((N_DEV,)),
        ],
        compiler_params=pltpu.CompilerParams(collective_id=0),
    )(Q, K, V, bt, lens2)
